# Initial kernel scaffold; baseline (speedup 1.0000x reference)
import jax
import jax.numpy as jnp
from jax.experimental import pallas as pl


def kernel(x, hyperedge_index, D_v_beta, D_e_beta_inv, D_e_alpha, D_v_alpha_inv, W1, b1, U1, c1, W2, b2, U2, c2):
    raise NotImplementedError("write your pallas kernel here")


# speedup vs baseline: 5.9834x; 5.9834x over previous
import functools

import jax
import jax.numpy as jnp
from jax import lax
from jax.experimental import pallas as pl
from jax.experimental.pallas import tpu as pltpu
from jax.experimental.pallas import tpu_sc as plsc

N = 10000
E = 10000
NNZ = 320000
NSUB = 16
DUMMY = N
ACC_ROWS = 10112
BN = 1000
NB = N // BN


def _make_sc_segsum(kj: int):
    zslc = ACC_ROWS // NSUB

    mesh = plsc.VectorSubcoreMesh(core_axis_name="c", subcore_axis_name="s")

    @functools.partial(
        pl.kernel,
        out_type=jax.ShapeDtypeStruct((2, 2, ACC_ROWS, 64), jnp.float32),
        mesh=mesh,
        compiler_params=pltpu.CompilerParams(use_tc_tiling_on_sc=False),
        scratch_types=[
            pltpu.VMEM((kj, 128), jnp.int32),
            pltpu.VMEM((kj, 128), jnp.int32),
            pltpu.VMEM((128, 64), jnp.float32),
            pltpu.VMEM_SHARED((ACC_ROWS, 64), jnp.float32),
        ],
    )
    def segsum(table_hbm, gidx_hbm, sidx_hbm, zeros_hbm, out_hbm,
               gi_v, si_v, rows_v, acc):
        c = lax.axis_index("c")
        w = lax.axis_index("s")
        pltpu.sync_copy(sidx_hbm.at[c, w], si_v)
        for h in range(2):
            pltpu.sync_copy(zeros_hbm.at[pl.ds(w * zslc, zslc)],
                            acc.at[pl.ds(w * zslc, zslc)])
            pltpu.sync_copy(gidx_hbm.at[c, h, w], gi_v)
            plsc.subcore_barrier()

            def body(j, carry):
                pltpu.sync_copy(table_hbm.at[gi_v.at[j]], rows_v)
                pltpu.sync_copy(rows_v, acc.at[si_v.at[j]], add=True)
                return carry

            lax.fori_loop(0, kj, body, 0)
            plsc.subcore_barrier()
            pltpu.sync_copy(acc.at[pl.ds(w * zslc, zslc)],
                            out_hbm.at[c, h, pl.ds(w * zslc, zslc)])

    return segsum


_sc_segsum_full = _make_sc_segsum(157)
_sc_segsum_half = _make_sc_segsum(79)


def _pad_idx(idx, n_sc, fill):
    per = idx.shape[0] // (n_sc * NSUB)
    kj = -(-per // 128)
    a = idx.reshape(n_sc, NSUB, per)
    a = jnp.pad(a, ((0, 0), (0, 0), (0, kj * 128 - per)), constant_values=fill)
    return a.reshape(n_sc, NSUB, kj, 128)


def _quarter_gidx(base, rows):
    return jnp.stack([
        jnp.stack([2 * (c * rows + base) + h for h in range(2)])
        for c in range(2)
    ])


def _vspec():
    return pl.BlockSpec((BN, 1), lambda i: (i, 0))


def _qspec():
    return pl.BlockSpec((2, 2, BN, 64), lambda i: (0, 0, i, 0))


def _split_spec():
    return pl.BlockSpec((2, BN, 128), lambda i: (0, i, 0))


def _cat(q_ref, c):
    return jnp.concatenate([q_ref[c, 0], q_ref[c, 1]], axis=1)


def _tc1_body(x_ref, w_ref, b_ref, dvb_ref, out_ref):
    h = jnp.dot(x_ref[...], w_ref[...], preferred_element_type=jnp.float32)
    h = (h + b_ref[...]) * dvb_ref[...]
    out_ref[0] = h[:, :128]
    out_ref[1] = h[:, 128:]


_tc1 = pl.pallas_call(
    _tc1_body,
    grid=(NB,),
    in_specs=[
        pl.BlockSpec((BN, 128), lambda i: (i, 0)),
        pl.BlockSpec((128, 256), lambda i: (0, 0)),
        pl.BlockSpec((1, 256), lambda i: (0, 0)),
        _vspec(),
    ],
    out_specs=_split_spec(),
    out_shape=jax.ShapeDtypeStruct((2, N, 128), jnp.float32),
)


def _tc2_body(a_ref, debi_ref, dea_ref, u_ref, c_ref, out_ref):
    debi = debi_ref[...]
    t0 = jax.nn.relu(_cat(a_ref, 0) * debi)
    t1 = jax.nn.relu(_cat(a_ref, 1) * debi)
    o = jnp.dot(t0, u_ref[:128, :], preferred_element_type=jnp.float32)
    o += jnp.dot(t1, u_ref[128:, :], preferred_element_type=jnp.float32)
    o = (o + c_ref[...]) * dea_ref[...]
    out_ref[0] = o[:, :128]
    out_ref[1] = o[:, 128:]


_tc2 = pl.pallas_call(
    _tc2_body,
    grid=(NB,),
    in_specs=[
        _qspec(),
        _vspec(),
        _vspec(),
        pl.BlockSpec((256, 256), lambda i: (0, 0)),
        pl.BlockSpec((1, 256), lambda i: (0, 0)),
    ],
    out_specs=_split_spec(),
    out_shape=jax.ShapeDtypeStruct((2, E, 128), jnp.float32),
)


def _tc3_body(a_ref, dvai_ref, dvb_ref, w_ref, b_ref, out_ref):
    dvai = dvai_ref[...]
    t0 = jax.nn.relu(_cat(a_ref, 0) * dvai)
    t1 = jax.nn.relu(_cat(a_ref, 1) * dvai)
    h = jnp.dot(t0, w_ref[:128, :], preferred_element_type=jnp.float32)
    h += jnp.dot(t1, w_ref[128:, :], preferred_element_type=jnp.float32)
    h = (h + b_ref[...]) * dvb_ref[...]
    out_ref[0] = h[:, :128]
    out_ref[1] = h[:, 128:]


_tc3 = pl.pallas_call(
    _tc3_body,
    grid=(NB,),
    in_specs=[
        _qspec(),
        _vspec(),
        _vspec(),
        pl.BlockSpec((256, 256), lambda i: (0, 0)),
        pl.BlockSpec((1, 256), lambda i: (0, 0)),
    ],
    out_specs=_split_spec(),
    out_shape=jax.ShapeDtypeStruct((2, N, 128), jnp.float32),
)


def _tc4_body(a_ref, debi_ref, dea_ref, u_ref, c_ref, e_ref, o_ref):
    debi = debi_ref[...]
    e0 = _cat(a_ref, 0) * debi
    e1 = _cat(a_ref, 1) * debi
    e_ref[:, :128] = e0
    e_ref[:, 128:] = e1
    o = jnp.dot(jax.nn.relu(e0), u_ref[:128, :],
                preferred_element_type=jnp.float32)
    o += jnp.dot(jax.nn.relu(e1), u_ref[128:, :],
                 preferred_element_type=jnp.float32)
    o_ref[...] = (o + c_ref[...]) * dea_ref[...]


_tc4 = pl.pallas_call(
    _tc4_body,
    grid=(NB,),
    in_specs=[
        _qspec(),
        _vspec(),
        _vspec(),
        pl.BlockSpec((256, 128), lambda i: (0, 0)),
        pl.BlockSpec((1, 128), lambda i: (0, 0)),
    ],
    out_specs=[
        pl.BlockSpec((BN, 256), lambda i: (i, 0)),
        pl.BlockSpec((BN, 128), lambda i: (i, 0)),
    ],
    out_shape=[
        jax.ShapeDtypeStruct((E, 256), jnp.float32),
        jax.ShapeDtypeStruct((E, 128), jnp.float32),
    ],
)


def _tc5_body(p_ref, dvai_ref, out_ref):
    lo = p_ref[0, 0] + p_ref[1, 0]
    hi = p_ref[0, 1] + p_ref[1, 1]
    out_ref[...] = jnp.concatenate([lo, hi], axis=1) * dvai_ref[...]


_tc5 = pl.pallas_call(
    _tc5_body,
    grid=(NB,),
    in_specs=[_qspec(), _vspec()],
    out_specs=pl.BlockSpec((BN, 128), lambda i: (i, 0)),
    out_shape=jax.ShapeDtypeStruct((N, 128), jnp.float32),
)


def kernel(x, hyperedge_index, D_v_beta, D_e_beta_inv, D_e_alpha, D_v_alpha_inv,
           W1, b1, U1, c1, W2, b2, U2, c2):
    src = hyperedge_index[0]
    dst = hyperedge_index[1]

    src_g = _pad_idx(src, 1, 0)[0]
    src_s = _pad_idx(src, 1, DUMMY)[0]
    dst_g = _pad_idx(dst, 1, 0)[0]
    dst_s = _pad_idx(dst, 1, DUMMY)[0]
    gidx_a = _quarter_gidx(src_g, N)
    sidx_a = jnp.stack([dst_s, dst_s])
    gidx_b = _quarter_gidx(dst_g, E)
    sidx_b = jnp.stack([src_s, src_s])
    dst_g2 = _pad_idx(dst, 2, 0)
    gidx_d = jnp.stack([2 * dst_g2[c] + jnp.arange(2).reshape(2, 1, 1, 1)
                        for c in range(2)])
    sidx_d = _pad_idx(src, 2, DUMMY)

    zeros = jnp.zeros((ACC_ROWS, 64), jnp.float32)
    dvb = D_v_beta.reshape(N, 1)
    debi = D_e_beta_inv.reshape(E, 1)
    dea = D_e_alpha.reshape(E, 1)
    dvai = D_v_alpha_inv.reshape(N, 1)

    h = _tc1(x, W1, b1.reshape(1, 256), dvb)
    ae = _sc_segsum_full(h.reshape(4 * N, 64), gidx_a, sidx_a, zeros)
    o = _tc2(ae, debi, dea, U1, c1.reshape(1, 256))
    av = _sc_segsum_full(o.reshape(4 * E, 64), gidx_b, sidx_b, zeros)
    h2 = _tc3(av, dvai, dvb, W2, b2.reshape(1, 256))
    ae2 = _sc_segsum_full(h2.reshape(4 * N, 64), gidx_a, sidx_a, zeros)
    e_out, o2 = _tc4(ae2, debi, dea, U2, c2.reshape(1, 128))
    p = _sc_segsum_half(o2.reshape(2 * E, 64), gidx_d, sidx_d, zeros)
    out = _tc5(p, dvai)
    return (out, e_out)
